# baseline (speedup 1.0000x reference)
import functools

import jax
import jax.numpy as jnp
from jax import lax
from jax.experimental import pallas as pl
from jax.experimental.pallas import tpu as pltpu
from jax.experimental.pallas import tpu_sc as plsc

N = 10000
NP = 10240
E = 320000
G = 256
D = 128

_KC = 80
_NSUB = 16
_NCORE = 2
_NCH = 128
_EPT = _NCH * _KC
_EPAD = _EPT * _NCORE * _NSUB
_RPT = NP // _NSUB


@functools.lru_cache(maxsize=None)
def _sc_kernels():
    mesh = plsc.VectorSubcoreMesh(core_axis_name="c", subcore_axis_name="s")

    @functools.partial(
        pl.kernel,
        mesh=mesh,
        out_type=jax.ShapeDtypeStruct((_NCORE, NP), jnp.float32),
        scratch_types=[
            pltpu.VMEM((_NCH, _KC), jnp.int32),
            pltpu.VMEM((_KC,), jnp.float32),
            pltpu.VMEM_SHARED((NP,), jnp.float32),
            pltpu.SemaphoreType.DMA,
        ],
    )
    def sc_degree(dst_hbm, zeros_hbm, out_hbm, di_v, ones_v, acc, sem):
        c = lax.axis_index("c")
        s = lax.axis_index("s")
        for i in range(_KC // 16):
            ones_v[pl.ds(i * 16, 16)] = jnp.ones((16,), jnp.float32)

        @pl.when(s == 0)
        def _zero():
            pltpu.sync_copy(zeros_hbm, acc)

        row0 = (c * _NSUB + s) * _NCH
        pltpu.sync_copy(dst_hbm.at[pl.ds(row0, _NCH)], di_v)
        plsc.subcore_barrier()

        def body(k, carry):
            j = pl.multiple_of(k * 8, 8)
            for u in range(8):
                pltpu.async_copy(ones_v, acc.at[di_v.at[j + u]], sem, add=True)
            for u in range(8):
                pltpu.make_async_copy(ones_v, acc.at[di_v.at[j + u]], sem).wait()
            return carry

        lax.fori_loop(0, _NCH // 8, body, 0)
        plsc.subcore_barrier()
        pltpu.sync_copy(acc.at[pl.ds(s * _RPT, _RPT)],
                        out_hbm.at[c, pl.ds(s * _RPT, _RPT)])

    @functools.partial(
        pl.kernel,
        mesh=mesh,
        out_type=jax.ShapeDtypeStruct((_NCORE, NP, D), jnp.float32),
        scratch_types=[
            pltpu.VMEM((_NCH // 2, _KC), jnp.int32),
            pltpu.VMEM((_NCH // 2, _KC), jnp.int32),
            pltpu.VMEM((_KC, D), jnp.float32),
            pltpu.VMEM((_KC, D), jnp.float32),
            pltpu.VMEM_SHARED((NP, D), jnp.float32),
            pltpu.SemaphoreType.DMA,
            pltpu.SemaphoreType.DMA,
        ],
    )
    def sc_propagate(table_hbm, src_hbm, dst_hbm, zeros_hbm, out_hbm,
                     si_v, di_v, rows0, rows1, acc, sem0, sem1):
        c = lax.axis_index("c")
        s = lax.axis_index("s")
        hch = _NCH // 2

        @pl.when(s == 0)
        def _zero():
            pltpu.sync_copy(zeros_hbm, acc)

        row0 = (c * _NSUB + s) * _NCH
        plsc.subcore_barrier()

        for h in range(2):
            pltpu.sync_copy(src_hbm.at[pl.ds(row0 + h * hch, hch)], si_v)
            pltpu.sync_copy(dst_hbm.at[pl.ds(row0 + h * hch, hch)], di_v)
            pltpu.async_copy(table_hbm.at[si_v.at[0]], rows0, sem0)

            def body(k, carry):
                j = pl.multiple_of(k * 2, 2)
                pltpu.make_async_copy(table_hbm.at[si_v.at[j]], rows0, sem0).wait()
                pltpu.async_copy(table_hbm.at[si_v.at[j + 1]], rows1, sem1)
                pltpu.sync_copy(rows0, acc.at[di_v.at[j]], add=True)
                pltpu.make_async_copy(table_hbm.at[si_v.at[j + 1]], rows1,
                                      sem1).wait()

                @pl.when(j + 2 < hch)
                def _prefetch():
                    pltpu.async_copy(table_hbm.at[si_v.at[j + 2]], rows0, sem0)

                pltpu.sync_copy(rows1, acc.at[di_v.at[j + 1]], add=True)
                return carry

            lax.fori_loop(0, hch // 2, body, 0)
        plsc.subcore_barrier()
        pltpu.sync_copy(acc.at[pl.ds(s * _RPT, _RPT)],
                        out_hbm.at[c, pl.ds(s * _RPT, _RPT)])

    return sc_degree, sc_propagate


def _tc_layer1(x, W1, degp):
    def body(x_ref, w_ref, degp_ref, scaled_ref, dinv_ref):
        dp = degp_ref[...]
        deg = dp[0, :N] + dp[1, :N] + 1.0
        dinv = lax.rsqrt(deg)
        dinv_ref[...] = dinv
        hw = jnp.dot(x_ref[...], w_ref[...], preferred_element_type=jnp.float32)
        scaled_ref[...] = hw * dinv

    return pl.pallas_call(
        body,
        out_shape=(jax.ShapeDtypeStruct((N, D), jnp.float32),
                   jax.ShapeDtypeStruct((N, 1), jnp.float32)),
    )(x, W1, degp)


def _tc_mid(tp, scaled_prev, dinv, b_prev, W):
    def body(tp_ref, sc_ref, dinv_ref, b_ref, w_ref, out_ref):
        tp_ = tp_ref[...]
        t = tp_[0, :N] + tp_[1, :N]
        dinv_ = dinv_ref[...]
        h = jnp.maximum(dinv_ * (t + sc_ref[...]) + b_ref[...], 0.0)
        out_ref[...] = jnp.dot(h, w_ref[...],
                               preferred_element_type=jnp.float32) * dinv_

    return pl.pallas_call(
        body,
        out_shape=jax.ShapeDtypeStruct((N, D), jnp.float32),
    )(tp, scaled_prev, dinv, b_prev, W)


def _tc_final(tp, scaled_prev, dinv, b_prev, batch2d,
              Wm1, bm1, Wm2, bm2, Wh1, bh1, Wh2, bh2):
    def body(tp_ref, sc_ref, dinv_ref, b_ref, batch_ref,
             wm1_ref, bm1_ref, wm2_ref, bm2_ref,
             wh1_ref, bh1_ref, wh2_ref, bh2_ref, out_ref):
        tp_ = tp_ref[...]
        t = tp_[0, :N] + tp_[1, :N]
        h = dinv_ref[...] * (t + sc_ref[...]) + b_ref[...]
        gids = lax.broadcasted_iota(jnp.int32, (N, G), 1)
        onehot = (batch_ref[...] == gids).astype(jnp.float32)
        dn = (((0,), (0,)), ((), ()))
        sums = lax.dot_general(onehot, h, dn,
                               preferred_element_type=jnp.float32)
        counts = lax.dot_general(onehot, jnp.ones((N, 1), jnp.float32), dn,
                                 preferred_element_type=jnp.float32)
        pooled = sums / jnp.maximum(counts, 1.0)
        z = jnp.maximum(jnp.dot(pooled, wm1_ref[...],
                                preferred_element_type=jnp.float32)
                        + bm1_ref[...], 0.0)
        z = jnp.maximum(jnp.dot(z, wm2_ref[...],
                                preferred_element_type=jnp.float32)
                        + bm2_ref[...], 0.0)
        r = jnp.maximum(jnp.dot(z, wh1_ref[...],
                                preferred_element_type=jnp.float32)
                        + bh1_ref[...], 0.0)
        out_ref[...] = jnp.dot(r, wh2_ref[...],
                               preferred_element_type=jnp.float32) + bh2_ref[...]

    return pl.pallas_call(
        body,
        out_shape=jax.ShapeDtypeStruct((G, D), jnp.float32),
    )(tp, scaled_prev, dinv, b_prev, batch2d,
      Wm1, bm1, Wm2, bm2, Wh1, bh1, Wh2, bh2)


def kernel(x, edge_index, batch, W1, b1, W2, b2, W3, b3,
           Wm1, bm1, Wm2, bm2, Wh1, bh1, Wh2, bh2):
    sc_degree, sc_propagate = _sc_kernels()
    npad = _EPAD - E
    pad_src = jnp.zeros((npad,), jnp.int32)
    pad_dst = N + (jnp.arange(npad, dtype=jnp.int32) % (NP - N))
    src = jnp.concatenate([edge_index[0].astype(jnp.int32), pad_src])
    dst = jnp.concatenate([edge_index[1].astype(jnp.int32), pad_dst])
    src = src.reshape(_EPAD // _KC, _KC)
    dst = dst.reshape(_EPAD // _KC, _KC)
    z1 = jnp.zeros((NP,), jnp.float32)
    z2 = jnp.zeros((NP, D), jnp.float32)

    degp = sc_degree(dst, z1).reshape(_NCORE, NP, 1)
    scaled1, dinv = _tc_layer1(x, W1, degp)
    t1 = sc_propagate(scaled1, src, dst, z2)
    scaled2 = _tc_mid(t1, scaled1, dinv, b1.reshape(1, D), W2)
    t2 = sc_propagate(scaled2, src, dst, z2)
    scaled3 = _tc_mid(t2, scaled2, dinv, b2.reshape(1, D), W3)
    t3 = sc_propagate(scaled3, src, dst, z2)
    return _tc_final(t3, scaled3, dinv, b3.reshape(1, D),
                     batch.astype(jnp.int32).reshape(N, 1),
                     Wm1, bm1.reshape(1, -1), Wm2, bm2.reshape(1, -1),
                     Wh1, bh1.reshape(1, -1), Wh2, bh2.reshape(1, -1))


# speedup vs baseline: 2.6152x; 2.6152x over previous
import functools

import jax
import jax.numpy as jnp
from jax import lax
from jax.experimental import pallas as pl
from jax.experimental.pallas import tpu as pltpu
from jax.experimental.pallas import tpu_sc as plsc

N = 10000
NP = 10240
E = 320000
G = 256
D = 128

_KC = 80
_NSUB = 16
_NCORE = 2
_NCH = 128
_EPT = _NCH * _KC
_EPAD = _EPT * _NCORE * _NSUB
_RPT = NP // _NSUB


@functools.lru_cache(maxsize=None)
def _sc_kernels():
    mesh = plsc.VectorSubcoreMesh(core_axis_name="c", subcore_axis_name="s")

    @functools.partial(
        pl.kernel,
        mesh=mesh,
        out_type=jax.ShapeDtypeStruct((_NCORE, NP), jnp.float32),
        scratch_types=[
            pltpu.VMEM((_NCH, _KC), jnp.int32),
            pltpu.VMEM((_KC,), jnp.float32),
            pltpu.VMEM_SHARED((NP,), jnp.float32),
            pltpu.SemaphoreType.DMA,
        ],
    )
    def sc_degree(dst_hbm, zeros_hbm, out_hbm, di_v, ones_v, acc, sem):
        c = lax.axis_index("c")
        s = lax.axis_index("s")
        for i in range(_KC // 16):
            ones_v[pl.ds(i * 16, 16)] = jnp.ones((16,), jnp.float32)

        @pl.when(s == 0)
        def _zero():
            pltpu.sync_copy(zeros_hbm, acc)

        row0 = (c * _NSUB + s) * _NCH
        pltpu.sync_copy(dst_hbm.at[pl.ds(row0, _NCH)], di_v)
        plsc.subcore_barrier()

        def body(k, carry):
            j = pl.multiple_of(k * 8, 8)
            for u in range(8):
                pltpu.async_copy(ones_v, acc.at[di_v.at[j + u]], sem, add=True)
            for u in range(8):
                pltpu.make_async_copy(ones_v, acc.at[di_v.at[j + u]], sem).wait()
            return carry

        lax.fori_loop(0, _NCH // 8, body, 0)
        plsc.subcore_barrier()
        pltpu.sync_copy(acc.at[pl.ds(s * _RPT, _RPT)],
                        out_hbm.at[c, pl.ds(s * _RPT, _RPT)])

    @functools.partial(
        pl.kernel,
        mesh=mesh,
        out_type=jax.ShapeDtypeStruct((_NCORE, NP, D), jnp.float32),
        scratch_types=[
            pltpu.VMEM((_NCH // 2, _KC), jnp.int32),
            pltpu.VMEM((_NCH // 2, _KC), jnp.int32),
            pltpu.VMEM((_KC, D), jnp.float32),
            pltpu.VMEM((_KC, D), jnp.float32),
            pltpu.VMEM_SHARED((NP, D), jnp.float32),
            pltpu.SemaphoreType.DMA,
            pltpu.SemaphoreType.DMA,
        ],
    )
    def sc_propagate(table_hbm, src_hbm, dst_hbm, zeros_hbm, out_hbm,
                     si_v, di_v, rows0, rows1, acc, sem0, sem1):
        c = lax.axis_index("c")
        s = lax.axis_index("s")
        hch = _NCH // 2

        @pl.when(s == 0)
        def _zero():
            pltpu.sync_copy(zeros_hbm, acc)

        row0 = (c * _NSUB + s) * _NCH
        plsc.subcore_barrier()

        for h in range(2):
            pltpu.sync_copy(src_hbm.at[pl.ds(row0 + h * hch, hch)], si_v)
            pltpu.sync_copy(dst_hbm.at[pl.ds(row0 + h * hch, hch)], di_v)
            pltpu.async_copy(table_hbm.at[si_v.at[0]], rows0, sem0)

            def body(k, carry):
                j = pl.multiple_of(k * 2, 2)
                pltpu.make_async_copy(table_hbm.at[si_v.at[j]], rows0, sem0).wait()
                pltpu.async_copy(table_hbm.at[si_v.at[j + 1]], rows1, sem1)
                pltpu.sync_copy(rows0, acc.at[di_v.at[j]], add=True)
                pltpu.make_async_copy(table_hbm.at[si_v.at[j + 1]], rows1,
                                      sem1).wait()

                @pl.when(j + 2 < hch)
                def _prefetch():
                    pltpu.async_copy(table_hbm.at[si_v.at[j + 2]], rows0, sem0)

                pltpu.sync_copy(rows1, acc.at[di_v.at[j + 1]], add=True)
                return carry

            lax.fori_loop(0, hch // 2, body, 0)
        plsc.subcore_barrier()
        pltpu.sync_copy(acc.at[pl.ds(s * _RPT, _RPT)],
                        out_hbm.at[c, pl.ds(s * _RPT, _RPT)])

    return sc_degree, sc_propagate


def _tc_layer1(x, W1, degp):
    def body(x_ref, w_ref, degp_ref, scaled_ref, dinv_ref):
        dp = degp_ref[...]
        deg = dp[0, :N] + dp[1, :N] + 1.0
        dinv = lax.rsqrt(deg)
        dinv_ref[...] = dinv
        hw = jnp.dot(x_ref[...], w_ref[...], preferred_element_type=jnp.float32)
        scaled_ref[...] = hw * dinv

    return pl.pallas_call(
        body,
        out_shape=(jax.ShapeDtypeStruct((N, D), jnp.float32),
                   jax.ShapeDtypeStruct((N, 1), jnp.float32)),
    )(x, W1, degp)


def _tc_mid(tp, scaled_prev, dinv, b_prev, W):
    def body(tp_ref, sc_ref, dinv_ref, b_ref, w_ref, out_ref):
        tp_ = tp_ref[...]
        t = tp_[0, :N] + tp_[1, :N]
        dinv_ = dinv_ref[...]
        h = jnp.maximum(dinv_ * (t + sc_ref[...]) + b_ref[...], 0.0)
        out_ref[...] = jnp.dot(h, w_ref[...],
                               preferred_element_type=jnp.float32) * dinv_

    return pl.pallas_call(
        body,
        out_shape=jax.ShapeDtypeStruct((N, D), jnp.float32),
    )(tp, scaled_prev, dinv, b_prev, W)


def _tc_final(tp, scaled_prev, dinv, b_prev, batch2d,
              Wm1, bm1, Wm2, bm2, Wh1, bh1, Wh2, bh2):
    def body(tp_ref, sc_ref, dinv_ref, b_ref, batch_ref,
             wm1_ref, bm1_ref, wm2_ref, bm2_ref,
             wh1_ref, bh1_ref, wh2_ref, bh2_ref, out_ref):
        tp_ = tp_ref[...]
        t = tp_[0, :N] + tp_[1, :N]
        h = dinv_ref[...] * (t + sc_ref[...]) + b_ref[...]
        gids = lax.broadcasted_iota(jnp.int32, (N, G), 1)
        onehot = (batch_ref[...] == gids).astype(jnp.float32)
        dn = (((0,), (0,)), ((), ()))
        sums = lax.dot_general(onehot, h, dn,
                               preferred_element_type=jnp.float32)
        counts = lax.dot_general(onehot, jnp.ones((N, 1), jnp.float32), dn,
                                 preferred_element_type=jnp.float32)
        pooled = sums / jnp.maximum(counts, 1.0)
        z = jnp.maximum(jnp.dot(pooled, wm1_ref[...],
                                preferred_element_type=jnp.float32)
                        + bm1_ref[...], 0.0)
        z = jnp.maximum(jnp.dot(z, wm2_ref[...],
                                preferred_element_type=jnp.float32)
                        + bm2_ref[...], 0.0)
        r = jnp.maximum(jnp.dot(z, wh1_ref[...],
                                preferred_element_type=jnp.float32)
                        + bh1_ref[...], 0.0)
        out_ref[...] = jnp.dot(r, wh2_ref[...],
                               preferred_element_type=jnp.float32) + bh2_ref[...]

    return pl.pallas_call(
        body,
        out_shape=jax.ShapeDtypeStruct((G, D), jnp.float32),
    )(tp, scaled_prev, dinv, b_prev, batch2d,
      Wm1, bm1, Wm2, bm2, Wh1, bh1, Wh2, bh2)


def kernel(x, edge_index, batch, W1, b1, W2, b2, W3, b3,
           Wm1, bm1, Wm2, bm2, Wh1, bh1, Wh2, bh2):
    sc_degree, sc_propagate = _sc_kernels()
    ntile = _NCORE * _NSUB
    nreal = E // (_KC * ntile)
    npad_c = _NCH - nreal
    pad = jnp.arange(npad_c * _KC, dtype=jnp.int32).reshape(1, npad_c, _KC)
    src3 = edge_index[0].astype(jnp.int32).reshape(ntile, nreal, _KC)
    dst3 = edge_index[1].astype(jnp.int32).reshape(ntile, nreal, _KC)
    src = jnp.concatenate(
        [src3, jnp.broadcast_to(pad, (ntile, npad_c, _KC))], axis=1)
    dst = jnp.concatenate(
        [dst3, jnp.broadcast_to(pad + N, (ntile, npad_c, _KC))], axis=1)
    src = src.reshape(_EPAD // _KC, _KC)
    dst = dst.reshape(_EPAD // _KC, _KC)
    z1 = jnp.zeros((NP,), jnp.float32)
    z2 = jnp.zeros((NP, D), jnp.float32)

    degp = sc_degree(dst, z1).reshape(_NCORE, NP, 1)
    scaled1, dinv = _tc_layer1(x, W1, degp)
    t1 = sc_propagate(scaled1, src, dst, z2)
    scaled2 = _tc_mid(t1, scaled1, dinv, b1.reshape(1, D), W2)
    t2 = sc_propagate(scaled2, src, dst, z2)
    scaled3 = _tc_mid(t2, scaled2, dinv, b2.reshape(1, D), W3)
    t3 = sc_propagate(scaled3, src, dst, z2)
    return _tc_final(t3, scaled3, dinv, b3.reshape(1, D),
                     batch.astype(jnp.int32).reshape(N, 1),
                     Wm1, bm1.reshape(1, -1), Wm2, bm2.reshape(1, -1),
                     Wh1, bh1.reshape(1, -1), Wh2, bh2.reshape(1, -1))


# speedup vs baseline: 2.9664x; 1.1343x over previous
import functools

import jax
import jax.numpy as jnp
from jax import lax
from jax.experimental import pallas as pl
from jax.experimental.pallas import tpu as pltpu
from jax.experimental.pallas import tpu_sc as plsc

N = 10000
NP = 10240
E = 320000
G = 256
D = 128

_KC = 80
_NSUB = 16
_NCORE = 2
_NCH = 128
_EPT = _NCH * _KC
_EPAD = _EPT * _NCORE * _NSUB
_RPT = NP // _NSUB


@functools.lru_cache(maxsize=None)
def _sc_kernels():
    mesh = plsc.VectorSubcoreMesh(core_axis_name="c", subcore_axis_name="s")

    @functools.partial(
        pl.kernel,
        mesh=mesh,
        out_type=jax.ShapeDtypeStruct((_NCORE, NP), jnp.float32),
        scratch_types=[
            pltpu.VMEM((_NCH, _KC), jnp.int32),
            pltpu.VMEM((_KC,), jnp.float32),
            pltpu.VMEM_SHARED((NP,), jnp.float32),
            pltpu.SemaphoreType.DMA,
        ],
    )
    def sc_degree(dst_hbm, zeros_hbm, out_hbm, di_v, ones_v, acc, sem):
        c = lax.axis_index("c")
        s = lax.axis_index("s")
        for i in range(_KC // 16):
            ones_v[pl.ds(i * 16, 16)] = jnp.ones((16,), jnp.float32)

        @pl.when(s == 0)
        def _zero():
            pltpu.sync_copy(zeros_hbm, acc)

        row0 = (c * _NSUB + s) * _NCH
        pltpu.sync_copy(dst_hbm.at[pl.ds(row0, _NCH)], di_v)
        plsc.subcore_barrier()

        def body(k, carry):
            j = pl.multiple_of(k * 8, 8)
            for u in range(8):
                pltpu.async_copy(ones_v, acc.at[di_v.at[j + u]], sem, add=True)
            for u in range(8):
                pltpu.make_async_copy(ones_v, acc.at[di_v.at[j + u]], sem).wait()
            return carry

        lax.fori_loop(0, _NCH // 8, body, 0)
        plsc.subcore_barrier()
        pltpu.sync_copy(acc.at[pl.ds(s * _RPT, _RPT)],
                        out_hbm.at[c, pl.ds(s * _RPT, _RPT)])

    @functools.partial(
        pl.kernel,
        mesh=mesh,
        out_type=jax.ShapeDtypeStruct((_NCORE, NP, D), jnp.float32),
        scratch_types=[
            pltpu.VMEM((_KC, D), jnp.float32),
            pltpu.VMEM((_KC, D), jnp.float32),
            pltpu.VMEM((_KC, D), jnp.float32),
            pltpu.VMEM((_KC,), jnp.int32),
            pltpu.VMEM((_KC,), jnp.int32),
            pltpu.VMEM((_KC,), jnp.int32),
            pltpu.VMEM((_KC,), jnp.int32),
            pltpu.VMEM((_KC,), jnp.int32),
            pltpu.VMEM((_KC,), jnp.int32),
            pltpu.SemaphoreType.DMA,
            pltpu.SemaphoreType.DMA,
            pltpu.SemaphoreType.DMA,
            pltpu.SemaphoreType.DMA,
            pltpu.SemaphoreType.DMA,
            pltpu.SemaphoreType.DMA,
            pltpu.VMEM_SHARED((NP, D), jnp.float32),
        ],
    )
    def sc_propagate(table_hbm, src_hbm, dst_hbm, zeros_hbm, out_hbm,
                     r0, r1, r2, sb0, sb1, sb2, db0, db1, db2,
                     ig0, ig1, ig2, g0, g1, g2, acc):
        c = lax.axis_index("c")
        s = lax.axis_index("s")
        rows = (r0, r1, r2)
        sbuf = (sb0, sb1, sb2)
        dbuf = (db0, db1, db2)
        isem = (ig0, ig1, ig2)
        gsem = (g0, g1, g2)

        @pl.when(s == 0)
        def _zero():
            pltpu.sync_copy(zeros_hbm, acc)

        base = (c * _NSUB + s) * _EPT
        plsc.subcore_barrier()

        def idx_load(j, u):
            off = pl.multiple_of(base + j * _KC, 8)
            pltpu.async_copy(src_hbm.at[pl.ds(off, _KC)], sbuf[u], isem[u])
            pltpu.async_copy(dst_hbm.at[pl.ds(off, _KC)], dbuf[u], isem[u])

        def idx_wait(u):
            pltpu.make_async_copy(src_hbm.at[pl.ds(0, _KC)], sbuf[u],
                                  isem[u]).wait()
            pltpu.make_async_copy(dst_hbm.at[pl.ds(0, _KC)], dbuf[u],
                                  isem[u]).wait()

        idx_load(0, 0)
        idx_load(1, 1)
        idx_load(2, 2)
        idx_wait(0)
        pltpu.async_copy(table_hbm.at[sbuf[0]], rows[0], gsem[0])
        idx_wait(1)
        pltpu.async_copy(table_hbm.at[sbuf[1]], rows[1], gsem[1])

        def step(j, u):
            u2 = (u + 2) % 3

            @pl.when(j + 2 < _NCH)
            def _launch():
                idx_wait(u2)
                pltpu.async_copy(table_hbm.at[sbuf[u2]], rows[u2], gsem[u2])

            pltpu.make_async_copy(table_hbm.at[sbuf[u]], rows[u],
                                  gsem[u]).wait()
            pltpu.sync_copy(rows[u], acc.at[dbuf[u]], add=True)

            @pl.when(j + 3 < _NCH)
            def _next_idx():
                idx_load(j + 3, u)

        def body(k, carry):
            j = pl.multiple_of(k * 3, 3)
            step(j, 0)
            step(j + 1, 1)
            step(j + 2, 2)
            return carry

        lax.fori_loop(0, _NCH // 3 - 1, body, 0)
        for j in range(_NCH - 3 - (_NCH % 3), _NCH):
            step(j, j % 3)
        plsc.subcore_barrier()
        pltpu.sync_copy(acc.at[pl.ds(s * _RPT, _RPT)],
                        out_hbm.at[c, pl.ds(s * _RPT, _RPT)])

    return sc_degree, sc_propagate


def _tc_layer1(x, W1, degp):
    def body(x_ref, w_ref, degp_ref, scaled_ref, dinv_ref):
        dp = degp_ref[...]
        deg = dp[0, :N] + dp[1, :N] + 1.0
        dinv = lax.rsqrt(deg)
        dinv_ref[...] = dinv
        hw = jnp.dot(x_ref[...], w_ref[...], preferred_element_type=jnp.float32)
        scaled_ref[...] = hw * dinv

    return pl.pallas_call(
        body,
        out_shape=(jax.ShapeDtypeStruct((N, D), jnp.float32),
                   jax.ShapeDtypeStruct((N, 1), jnp.float32)),
    )(x, W1, degp)


def _tc_mid(tp, scaled_prev, dinv, b_prev, W):
    def body(tp_ref, sc_ref, dinv_ref, b_ref, w_ref, out_ref):
        tp_ = tp_ref[...]
        t = tp_[0, :N] + tp_[1, :N]
        dinv_ = dinv_ref[...]
        h = jnp.maximum(dinv_ * (t + sc_ref[...]) + b_ref[...], 0.0)
        out_ref[...] = jnp.dot(h, w_ref[...],
                               preferred_element_type=jnp.float32) * dinv_

    return pl.pallas_call(
        body,
        out_shape=jax.ShapeDtypeStruct((N, D), jnp.float32),
    )(tp, scaled_prev, dinv, b_prev, W)


def _tc_final(tp, scaled_prev, dinv, b_prev, batch2d,
              Wm1, bm1, Wm2, bm2, Wh1, bh1, Wh2, bh2):
    def body(tp_ref, sc_ref, dinv_ref, b_ref, batch_ref,
             wm1_ref, bm1_ref, wm2_ref, bm2_ref,
             wh1_ref, bh1_ref, wh2_ref, bh2_ref, out_ref):
        tp_ = tp_ref[...]
        t = tp_[0, :N] + tp_[1, :N]
        h = dinv_ref[...] * (t + sc_ref[...]) + b_ref[...]
        gids = lax.broadcasted_iota(jnp.int32, (N, G), 1)
        onehot = (batch_ref[...] == gids).astype(jnp.float32)
        dn = (((0,), (0,)), ((), ()))
        sums = lax.dot_general(onehot, h, dn,
                               preferred_element_type=jnp.float32)
        counts = lax.dot_general(onehot, jnp.ones((N, 1), jnp.float32), dn,
                                 preferred_element_type=jnp.float32)
        pooled = sums / jnp.maximum(counts, 1.0)
        z = jnp.maximum(jnp.dot(pooled, wm1_ref[...],
                                preferred_element_type=jnp.float32)
                        + bm1_ref[...], 0.0)
        z = jnp.maximum(jnp.dot(z, wm2_ref[...],
                                preferred_element_type=jnp.float32)
                        + bm2_ref[...], 0.0)
        r = jnp.maximum(jnp.dot(z, wh1_ref[...],
                                preferred_element_type=jnp.float32)
                        + bh1_ref[...], 0.0)
        out_ref[...] = jnp.dot(r, wh2_ref[...],
                               preferred_element_type=jnp.float32) + bh2_ref[...]

    return pl.pallas_call(
        body,
        out_shape=jax.ShapeDtypeStruct((G, D), jnp.float32),
    )(tp, scaled_prev, dinv, b_prev, batch2d,
      Wm1, bm1, Wm2, bm2, Wh1, bh1, Wh2, bh2)


def kernel(x, edge_index, batch, W1, b1, W2, b2, W3, b3,
           Wm1, bm1, Wm2, bm2, Wh1, bh1, Wh2, bh2):
    sc_degree, sc_propagate = _sc_kernels()
    ntile = _NCORE * _NSUB
    nreal = E // (_KC * ntile)
    npad_c = _NCH - nreal
    pad = jnp.arange(npad_c * _KC, dtype=jnp.int32).reshape(1, npad_c, _KC)
    src3 = edge_index[0].astype(jnp.int32).reshape(ntile, nreal, _KC)
    dst3 = edge_index[1].astype(jnp.int32).reshape(ntile, nreal, _KC)
    src = jnp.concatenate(
        [src3, jnp.broadcast_to(pad, (ntile, npad_c, _KC))], axis=1)
    dst = jnp.concatenate(
        [dst3, jnp.broadcast_to(pad + N, (ntile, npad_c, _KC))], axis=1)
    dst2d = dst.reshape(_EPAD // _KC, _KC)
    src = src.reshape(_EPAD)
    dst = dst.reshape(_EPAD)
    z1 = jnp.zeros((NP,), jnp.float32)
    z2 = jnp.zeros((NP, D), jnp.float32)

    degp = sc_degree(dst2d, z1).reshape(_NCORE, NP, 1)
    scaled1, dinv = _tc_layer1(x, W1, degp)
    t1 = sc_propagate(scaled1, src, dst, z2)
    scaled2 = _tc_mid(t1, scaled1, dinv, b1.reshape(1, D), W2)
    t2 = sc_propagate(scaled2, src, dst, z2)
    scaled3 = _tc_mid(t2, scaled2, dinv, b2.reshape(1, D), W3)
    t3 = sc_propagate(scaled3, src, dst, z2)
    return _tc_final(t3, scaled3, dinv, b3.reshape(1, D),
                     batch.astype(jnp.int32).reshape(N, 1),
                     Wm1, bm1.reshape(1, -1), Wm2, bm2.reshape(1, -1),
                     Wh1, bh1.reshape(1, -1), Wh2, bh2.reshape(1, -1))


# speedup vs baseline: 3.8676x; 1.3038x over previous
import functools

import jax
import jax.numpy as jnp
from jax import lax
from jax.experimental import pallas as pl
from jax.experimental.pallas import tpu as pltpu
from jax.experimental.pallas import tpu_sc as plsc

N = 10000
NP = 10240
E = 320000
G = 256
D = 128

_KC = 80
_NSUB = 16
_NCORE = 2
_NCH = 128
_EPT = _NCH * _KC
_EPAD = _EPT * _NCORE * _NSUB
_RPT = NP // _NSUB


@functools.lru_cache(maxsize=None)
def _sc_kernels():
    mesh = plsc.VectorSubcoreMesh(core_axis_name="c", subcore_axis_name="s")

    @functools.partial(
        pl.kernel,
        mesh=mesh,
        out_type=jax.ShapeDtypeStruct((_NCORE, NP), jnp.float32),
        scratch_types=[
            pltpu.VMEM((_NCH, _KC), jnp.int32),
            pltpu.VMEM((_KC,), jnp.float32),
            pltpu.VMEM_SHARED((NP,), jnp.float32),
            pltpu.SemaphoreType.DMA,
        ],
    )
    def sc_degree(dst_hbm, zeros_hbm, out_hbm, di_v, ones_v, acc, sem):
        c = lax.axis_index("c")
        s = lax.axis_index("s")
        for i in range(_KC // 16):
            ones_v[pl.ds(i * 16, 16)] = jnp.ones((16,), jnp.float32)

        @pl.when(s == 0)
        def _zero():
            pltpu.sync_copy(zeros_hbm, acc)

        row0 = (c * _NSUB + s) * _NCH
        pltpu.sync_copy(dst_hbm.at[pl.ds(row0, _NCH)], di_v)
        plsc.subcore_barrier()

        def body(k, carry):
            j = pl.multiple_of(k * 8, 8)
            for u in range(8):
                pltpu.async_copy(ones_v, acc.at[di_v.at[j + u]], sem, add=True)
            for u in range(8):
                pltpu.make_async_copy(ones_v, acc.at[di_v.at[j + u]], sem).wait()
            return carry

        lax.fori_loop(0, _NCH // 8, body, 0)
        plsc.subcore_barrier()
        pltpu.sync_copy(acc.at[pl.ds(s * _RPT, _RPT)],
                        out_hbm.at[c, pl.ds(s * _RPT, _RPT)])

    @functools.partial(
        pl.kernel,
        mesh=mesh,
        out_type=jax.ShapeDtypeStruct((_NCORE, NP, D), jnp.float32),
        scratch_types=[
            pltpu.VMEM((_KC, D), jnp.float32),
            pltpu.VMEM((_KC, D), jnp.float32),
            pltpu.VMEM((_KC, D), jnp.float32),
            pltpu.VMEM((6, _KC), jnp.int32),
            pltpu.VMEM((6, _KC), jnp.int32),
            pltpu.SemaphoreType.DMA,
            pltpu.SemaphoreType.DMA,
            pltpu.SemaphoreType.DMA,
            pltpu.SemaphoreType.DMA,
            pltpu.SemaphoreType.DMA,
            pltpu.SemaphoreType.DMA,
            pltpu.SemaphoreType.DMA,
            pltpu.SemaphoreType.DMA,
            pltpu.SemaphoreType.DMA,
            pltpu.VMEM_SHARED((NP, D), jnp.float32),
        ],
    )
    def sc_propagate(table_hbm, src_hbm, dst_hbm, zeros_hbm, out_hbm,
                     r0, r1, r2, sb, db,
                     i0, i1, i2, i3, i4, i5, g0, g1, g2, acc):
        c = lax.axis_index("c")
        s = lax.axis_index("s")
        rows = (r0, r1, r2)
        isem = (i0, i1, i2, i3, i4, i5)
        gsem = (g0, g1, g2)

        @pl.when(s == 0)
        def _zero():
            pltpu.sync_copy(zeros_hbm, acc)

        base = (c * _NSUB + s) * _EPT
        plsc.subcore_barrier()

        def idx_load(j, u6):
            off = pl.multiple_of(base + j * _KC, 8)
            pltpu.async_copy(src_hbm.at[pl.ds(off, _KC)], sb.at[u6], isem[u6])
            pltpu.async_copy(dst_hbm.at[pl.ds(off, _KC)], db.at[u6], isem[u6])

        def idx_wait(u6):
            pltpu.make_async_copy(src_hbm.at[pl.ds(0, _KC)], sb.at[u6],
                                  isem[u6]).wait()
            pltpu.make_async_copy(dst_hbm.at[pl.ds(0, _KC)], db.at[u6],
                                  isem[u6]).wait()

        for m in range(6):
            idx_load(m, m)
        idx_wait(0)
        pltpu.async_copy(table_hbm.at[sb.at[0]], rows[0], gsem[0])
        idx_wait(1)
        pltpu.async_copy(table_hbm.at[sb.at[1]], rows[1], gsem[1])

        def step(j, u3, u6):
            @pl.when(j + 2 < _NCH)
            def _launch():
                idx_wait((u6 + 2) % 6)
                pltpu.async_copy(table_hbm.at[sb.at[(u6 + 2) % 6]],
                                 rows[(u3 + 2) % 3], gsem[(u3 + 2) % 3])

            pltpu.make_async_copy(table_hbm.at[sb.at[u6]], rows[u3],
                                  gsem[u3]).wait()
            pltpu.sync_copy(rows[u3], acc.at[db.at[u6]], add=True)

            @pl.when(j + 6 < _NCH)
            def _next_idx():
                idx_load(j + 6, u6)

        def body(k, carry):
            j = pl.multiple_of(k * 6, 6)
            for u in range(6):
                step(j + u, u % 3, u)
            return carry

        nfull = _NCH // 6 - 1
        lax.fori_loop(0, nfull, body, 0)
        for j in range(nfull * 6, _NCH):
            step(j, j % 3, j % 6)
        plsc.subcore_barrier()
        pltpu.sync_copy(acc.at[pl.ds(s * _RPT, _RPT)],
                        out_hbm.at[c, pl.ds(s * _RPT, _RPT)])

    return sc_degree, sc_propagate


def _tc_layer1(x, W1, degp):
    def body(x_ref, w_ref, degp_ref, scaled_ref, dinv_ref):
        dp = degp_ref[...]
        deg = dp[0, :N] + dp[1, :N] + 1.0
        dinv = lax.rsqrt(deg)
        dinv_ref[...] = dinv
        hw = jnp.dot(x_ref[...], w_ref[...], preferred_element_type=jnp.float32)
        scaled_ref[...] = hw * dinv

    return pl.pallas_call(
        body,
        out_shape=(jax.ShapeDtypeStruct((N, D), jnp.float32),
                   jax.ShapeDtypeStruct((N, 1), jnp.float32)),
    )(x, W1, degp)


def _tc_mid(tp, scaled_prev, dinv, b_prev, W):
    def body(tp_ref, sc_ref, dinv_ref, b_ref, w_ref, out_ref):
        tp_ = tp_ref[...]
        t = tp_[0, :N] + tp_[1, :N]
        dinv_ = dinv_ref[...]
        h = jnp.maximum(dinv_ * (t + sc_ref[...]) + b_ref[...], 0.0)
        out_ref[...] = jnp.dot(h, w_ref[...],
                               preferred_element_type=jnp.float32) * dinv_

    return pl.pallas_call(
        body,
        out_shape=jax.ShapeDtypeStruct((N, D), jnp.float32),
    )(tp, scaled_prev, dinv, b_prev, W)


def _tc_final(tp, scaled_prev, dinv, b_prev, batch2d,
              Wm1, bm1, Wm2, bm2, Wh1, bh1, Wh2, bh2):
    def body(tp_ref, sc_ref, dinv_ref, b_ref, batch_ref,
             wm1_ref, bm1_ref, wm2_ref, bm2_ref,
             wh1_ref, bh1_ref, wh2_ref, bh2_ref, out_ref):
        tp_ = tp_ref[...]
        t = tp_[0, :N] + tp_[1, :N]
        h = dinv_ref[...] * (t + sc_ref[...]) + b_ref[...]
        gids = lax.broadcasted_iota(jnp.int32, (N, G), 1)
        onehot = (batch_ref[...] == gids).astype(jnp.float32)
        dn = (((0,), (0,)), ((), ()))
        sums = lax.dot_general(onehot, h, dn,
                               preferred_element_type=jnp.float32)
        counts = lax.dot_general(onehot, jnp.ones((N, 1), jnp.float32), dn,
                                 preferred_element_type=jnp.float32)
        pooled = sums / jnp.maximum(counts, 1.0)
        z = jnp.maximum(jnp.dot(pooled, wm1_ref[...],
                                preferred_element_type=jnp.float32)
                        + bm1_ref[...], 0.0)
        z = jnp.maximum(jnp.dot(z, wm2_ref[...],
                                preferred_element_type=jnp.float32)
                        + bm2_ref[...], 0.0)
        r = jnp.maximum(jnp.dot(z, wh1_ref[...],
                                preferred_element_type=jnp.float32)
                        + bh1_ref[...], 0.0)
        out_ref[...] = jnp.dot(r, wh2_ref[...],
                               preferred_element_type=jnp.float32) + bh2_ref[...]

    return pl.pallas_call(
        body,
        out_shape=jax.ShapeDtypeStruct((G, D), jnp.float32),
    )(tp, scaled_prev, dinv, b_prev, batch2d,
      Wm1, bm1, Wm2, bm2, Wh1, bh1, Wh2, bh2)


def kernel(x, edge_index, batch, W1, b1, W2, b2, W3, b3,
           Wm1, bm1, Wm2, bm2, Wh1, bh1, Wh2, bh2):
    sc_degree, sc_propagate = _sc_kernels()
    ntile = _NCORE * _NSUB
    nreal = E // (_KC * ntile)
    npad_c = _NCH - nreal
    pad = jnp.arange(npad_c * _KC, dtype=jnp.int32).reshape(1, npad_c, _KC)
    src3 = edge_index[0].astype(jnp.int32).reshape(ntile, nreal, _KC)
    dst3 = edge_index[1].astype(jnp.int32).reshape(ntile, nreal, _KC)
    src = jnp.concatenate(
        [src3, jnp.broadcast_to(pad, (ntile, npad_c, _KC))], axis=1)
    dst = jnp.concatenate(
        [dst3, jnp.broadcast_to(pad + N, (ntile, npad_c, _KC))], axis=1)
    dst2d = dst.reshape(_EPAD // _KC, _KC)
    src = src.reshape(_EPAD)
    dst = dst.reshape(_EPAD)
    z1 = jnp.zeros((NP,), jnp.float32)
    z2 = jnp.zeros((NP, D), jnp.float32)

    degp = sc_degree(dst2d, z1).reshape(_NCORE, NP, 1)
    scaled1, dinv = _tc_layer1(x, W1, degp)
    t1 = sc_propagate(scaled1, src, dst, z2)
    scaled2 = _tc_mid(t1, scaled1, dinv, b1.reshape(1, D), W2)
    t2 = sc_propagate(scaled2, src, dst, z2)
    scaled3 = _tc_mid(t2, scaled2, dinv, b2.reshape(1, D), W3)
    t3 = sc_propagate(scaled3, src, dst, z2)
    return _tc_final(t3, scaled3, dinv, b3.reshape(1, D),
                     batch.astype(jnp.int32).reshape(N, 1),
                     Wm1, bm1.reshape(1, -1), Wm2, bm2.reshape(1, -1),
                     Wh1, bh1.reshape(1, -1), Wh2, bh2.reshape(1, -1))


# speedup vs baseline: 3.9188x; 1.0132x over previous
import functools

import jax
import jax.numpy as jnp
from jax import lax
from jax.experimental import pallas as pl
from jax.experimental.pallas import tpu as pltpu
from jax.experimental.pallas import tpu_sc as plsc

N = 10000
NP = 10240
E = 320000
G = 256
D = 128

_KC = 80
_NSUB = 16
_NCORE = 2
_NCH = 128
_EPT = _NCH * _KC
_EPAD = _EPT * _NCORE * _NSUB
_RPT = NP // _NSUB


@functools.lru_cache(maxsize=None)
def _sc_kernels():
    mesh = plsc.VectorSubcoreMesh(core_axis_name="c", subcore_axis_name="s")

    @functools.partial(
        pl.kernel,
        mesh=mesh,
        out_type=jax.ShapeDtypeStruct((_NCORE, NP), jnp.float32),
        scratch_types=[
            pltpu.VMEM((_NCH, _KC), jnp.int32),
            pltpu.VMEM((_KC,), jnp.float32),
            pltpu.VMEM_SHARED((NP,), jnp.float32),
            pltpu.SemaphoreType.DMA,
        ],
    )
    def sc_degree(dst_hbm, zeros_hbm, out_hbm, di_v, ones_v, acc, sem):
        c = lax.axis_index("c")
        s = lax.axis_index("s")
        for i in range(_KC // 16):
            ones_v[pl.ds(i * 16, 16)] = jnp.ones((16,), jnp.float32)

        row0 = (c * _NSUB + s) * _NCH
        pltpu.sync_copy(dst_hbm.at[pl.ds(row0, _NCH)], di_v)
        pltpu.sync_copy(zeros_hbm.at[pl.ds(s * _RPT, _RPT)],
                        acc.at[pl.ds(s * _RPT, _RPT)])
        plsc.subcore_barrier()

        def body(k, carry):
            j = pl.multiple_of(k * 8, 8)
            for u in range(8):
                pltpu.async_copy(ones_v, acc.at[di_v.at[j + u]], sem, add=True)
            for u in range(8):
                pltpu.make_async_copy(ones_v, acc.at[di_v.at[j + u]], sem).wait()
            return carry

        lax.fori_loop(0, _NCH // 8, body, 0)
        plsc.subcore_barrier()
        pltpu.sync_copy(acc.at[pl.ds(s * _RPT, _RPT)],
                        out_hbm.at[c, pl.ds(s * _RPT, _RPT)])

    @functools.partial(
        pl.kernel,
        mesh=mesh,
        out_type=jax.ShapeDtypeStruct((_NCORE, NP, D), jnp.float32),
        scratch_types=[
            pltpu.VMEM((_KC, D), jnp.float32),
            pltpu.VMEM((_KC, D), jnp.float32),
            pltpu.VMEM((_KC, D), jnp.float32),
            pltpu.VMEM((6, _KC), jnp.int32),
            pltpu.VMEM((6, _KC), jnp.int32),
            pltpu.SemaphoreType.DMA,
            pltpu.SemaphoreType.DMA,
            pltpu.SemaphoreType.DMA,
            pltpu.SemaphoreType.DMA,
            pltpu.SemaphoreType.DMA,
            pltpu.SemaphoreType.DMA,
            pltpu.SemaphoreType.DMA,
            pltpu.SemaphoreType.DMA,
            pltpu.SemaphoreType.DMA,
            pltpu.VMEM_SHARED((NP, D), jnp.float32),
        ],
    )
    def sc_propagate(table_hbm, src_hbm, dst_hbm, zeros_hbm, out_hbm,
                     r0, r1, r2, sb, db,
                     i0, i1, i2, i3, i4, i5, g0, g1, g2, acc):
        c = lax.axis_index("c")
        s = lax.axis_index("s")
        rows = (r0, r1, r2)
        isem = (i0, i1, i2, i3, i4, i5)
        gsem = (g0, g1, g2)

        base = (c * _NSUB + s) * _EPT

        def idx_load(j, u6):
            off = pl.multiple_of(base + j * _KC, 8)
            pltpu.async_copy(src_hbm.at[pl.ds(off, _KC)], sb.at[u6], isem[u6])
            pltpu.async_copy(dst_hbm.at[pl.ds(off, _KC)], db.at[u6], isem[u6])

        def idx_wait(u6):
            pltpu.make_async_copy(src_hbm.at[pl.ds(0, _KC)], sb.at[u6],
                                  isem[u6]).wait()
            pltpu.make_async_copy(dst_hbm.at[pl.ds(0, _KC)], db.at[u6],
                                  isem[u6]).wait()

        for m in range(6):
            idx_load(m, m)
        idx_wait(0)
        pltpu.async_copy(table_hbm.at[sb.at[0]], rows[0], gsem[0])
        idx_wait(1)
        pltpu.async_copy(table_hbm.at[sb.at[1]], rows[1], gsem[1])
        pltpu.sync_copy(zeros_hbm.at[pl.ds(s * _RPT, _RPT)],
                        acc.at[pl.ds(s * _RPT, _RPT)])
        plsc.subcore_barrier()

        def step(j, u3, u6):
            @pl.when(j + 2 < _NCH)
            def _launch():
                idx_wait((u6 + 2) % 6)
                pltpu.async_copy(table_hbm.at[sb.at[(u6 + 2) % 6]],
                                 rows[(u3 + 2) % 3], gsem[(u3 + 2) % 3])

            pltpu.make_async_copy(table_hbm.at[sb.at[u6]], rows[u3],
                                  gsem[u3]).wait()
            pltpu.sync_copy(rows[u3], acc.at[db.at[u6]], add=True)

            @pl.when(j + 6 < _NCH)
            def _next_idx():
                idx_load(j + 6, u6)

        def body(k, carry):
            j = pl.multiple_of(k * 6, 6)
            for u in range(6):
                step(j + u, u % 3, u)
            return carry

        nfull = _NCH // 6 - 1
        lax.fori_loop(0, nfull, body, 0)
        for j in range(nfull * 6, _NCH):
            step(j, j % 3, j % 6)
        plsc.subcore_barrier()
        pltpu.sync_copy(acc.at[pl.ds(s * _RPT, _RPT)],
                        out_hbm.at[c, pl.ds(s * _RPT, _RPT)])

    return sc_degree, sc_propagate


def _tc_layer1(x, W1, degp):
    def body(x_ref, w_ref, degp_ref, scaled_ref, dinv_ref):
        dp = degp_ref[...]
        deg = dp[0, :N] + dp[1, :N] + 1.0
        dinv = lax.rsqrt(deg)
        dinv_ref[...] = dinv
        hw = jnp.dot(x_ref[...], w_ref[...], preferred_element_type=jnp.float32)
        scaled_ref[...] = hw * dinv

    return pl.pallas_call(
        body,
        out_shape=(jax.ShapeDtypeStruct((N, D), jnp.float32),
                   jax.ShapeDtypeStruct((N, 1), jnp.float32)),
    )(x, W1, degp)


def _tc_mid(tp, scaled_prev, dinv, b_prev, W):
    def body(tp_ref, sc_ref, dinv_ref, b_ref, w_ref, out_ref):
        tp_ = tp_ref[...]
        t = tp_[0, :N] + tp_[1, :N]
        dinv_ = dinv_ref[...]
        h = jnp.maximum(dinv_ * (t + sc_ref[...]) + b_ref[...], 0.0)
        out_ref[...] = jnp.dot(h, w_ref[...],
                               preferred_element_type=jnp.float32) * dinv_

    return pl.pallas_call(
        body,
        out_shape=jax.ShapeDtypeStruct((N, D), jnp.float32),
    )(tp, scaled_prev, dinv, b_prev, W)


def _tc_final(tp, scaled_prev, dinv, b_prev, batch2d,
              Wm1, bm1, Wm2, bm2, Wh1, bh1, Wh2, bh2):
    def body(tp_ref, sc_ref, dinv_ref, b_ref, batch_ref,
             wm1_ref, bm1_ref, wm2_ref, bm2_ref,
             wh1_ref, bh1_ref, wh2_ref, bh2_ref, out_ref):
        tp_ = tp_ref[...]
        t = tp_[0, :N] + tp_[1, :N]
        h = dinv_ref[...] * (t + sc_ref[...]) + b_ref[...]
        gids = lax.broadcasted_iota(jnp.int32, (N, G), 1)
        onehot = (batch_ref[...] == gids).astype(jnp.float32)
        dn = (((0,), (0,)), ((), ()))
        sums = lax.dot_general(onehot, h, dn,
                               preferred_element_type=jnp.float32)
        counts = lax.dot_general(onehot, jnp.ones((N, 1), jnp.float32), dn,
                                 preferred_element_type=jnp.float32)
        pooled = sums / jnp.maximum(counts, 1.0)
        z = jnp.maximum(jnp.dot(pooled, wm1_ref[...],
                                preferred_element_type=jnp.float32)
                        + bm1_ref[...], 0.0)
        z = jnp.maximum(jnp.dot(z, wm2_ref[...],
                                preferred_element_type=jnp.float32)
                        + bm2_ref[...], 0.0)
        r = jnp.maximum(jnp.dot(z, wh1_ref[...],
                                preferred_element_type=jnp.float32)
                        + bh1_ref[...], 0.0)
        out_ref[...] = jnp.dot(r, wh2_ref[...],
                               preferred_element_type=jnp.float32) + bh2_ref[...]

    return pl.pallas_call(
        body,
        out_shape=jax.ShapeDtypeStruct((G, D), jnp.float32),
    )(tp, scaled_prev, dinv, b_prev, batch2d,
      Wm1, bm1, Wm2, bm2, Wh1, bh1, Wh2, bh2)


def kernel(x, edge_index, batch, W1, b1, W2, b2, W3, b3,
           Wm1, bm1, Wm2, bm2, Wh1, bh1, Wh2, bh2):
    sc_degree, sc_propagate = _sc_kernels()
    ntile = _NCORE * _NSUB
    nreal = E // (_KC * ntile)
    npad_c = _NCH - nreal
    pad = jnp.arange(npad_c * _KC, dtype=jnp.int32).reshape(1, npad_c, _KC)
    src3 = edge_index[0].astype(jnp.int32).reshape(ntile, nreal, _KC)
    dst3 = edge_index[1].astype(jnp.int32).reshape(ntile, nreal, _KC)
    src = jnp.concatenate(
        [src3, jnp.broadcast_to(pad, (ntile, npad_c, _KC))], axis=1)
    dst = jnp.concatenate(
        [dst3, jnp.broadcast_to(pad + N, (ntile, npad_c, _KC))], axis=1)
    dst2d = dst.reshape(_EPAD // _KC, _KC)
    src = src.reshape(_EPAD)
    dst = dst.reshape(_EPAD)
    z1 = jnp.zeros((NP,), jnp.float32)
    z2 = jnp.zeros((NP, D), jnp.float32)

    degp = sc_degree(dst2d, z1).reshape(_NCORE, NP, 1)
    scaled1, dinv = _tc_layer1(x, W1, degp)
    t1 = sc_propagate(scaled1, src, dst, z2)
    scaled2 = _tc_mid(t1, scaled1, dinv, b1.reshape(1, D), W2)
    t2 = sc_propagate(scaled2, src, dst, z2)
    scaled3 = _tc_mid(t2, scaled2, dinv, b2.reshape(1, D), W3)
    t3 = sc_propagate(scaled3, src, dst, z2)
    return _tc_final(t3, scaled3, dinv, b3.reshape(1, D),
                     batch.astype(jnp.int32).reshape(N, 1),
                     Wm1, bm1.reshape(1, -1), Wm2, bm2.reshape(1, -1),
                     Wh1, bh1.reshape(1, -1), Wh2, bh2.reshape(1, -1))


# speedup vs baseline: 3.9690x; 1.0128x over previous
import functools

import jax
import jax.numpy as jnp
from jax import lax
from jax.experimental import pallas as pl
from jax.experimental.pallas import tpu as pltpu
from jax.experimental.pallas import tpu_sc as plsc

N = 10000
NP = 10240
E = 320000
G = 256
D = 128

_KC = 64
_NSUB = 16
_NCORE = 2
_NCH = 160
_EPT = _NCH * _KC
_EPAD = _EPT * _NCORE * _NSUB
_RPT = NP // _NSUB


@functools.lru_cache(maxsize=None)
def _sc_kernels():
    mesh = plsc.VectorSubcoreMesh(core_axis_name="c", subcore_axis_name="s")

    @functools.partial(
        pl.kernel,
        mesh=mesh,
        out_type=jax.ShapeDtypeStruct((_NCORE, NP), jnp.float32),
        scratch_types=[
            pltpu.VMEM((_NCH, _KC), jnp.int32),
            pltpu.VMEM((_KC,), jnp.float32),
            pltpu.VMEM_SHARED((NP,), jnp.float32),
            pltpu.SemaphoreType.DMA,
        ],
    )
    def sc_degree(dst_hbm, zeros_hbm, out_hbm, di_v, ones_v, acc, sem):
        c = lax.axis_index("c")
        s = lax.axis_index("s")
        for i in range(_KC // 16):
            ones_v[pl.ds(i * 16, 16)] = jnp.ones((16,), jnp.float32)

        row0 = (c * _NSUB + s) * _NCH
        pltpu.sync_copy(dst_hbm.at[pl.ds(row0, _NCH)], di_v)
        pltpu.sync_copy(zeros_hbm.at[pl.ds(s * _RPT, _RPT)],
                        acc.at[pl.ds(s * _RPT, _RPT)])
        plsc.subcore_barrier()

        def body(k, carry):
            j = pl.multiple_of(k * 8, 8)
            for u in range(8):
                pltpu.async_copy(ones_v, acc.at[di_v.at[j + u]], sem, add=True)
            for u in range(8):
                pltpu.make_async_copy(ones_v, acc.at[di_v.at[j + u]], sem).wait()
            return carry

        lax.fori_loop(0, _NCH // 8, body, 0)
        plsc.subcore_barrier()
        pltpu.sync_copy(acc.at[pl.ds(s * _RPT, _RPT)],
                        out_hbm.at[c, pl.ds(s * _RPT, _RPT)])

    @functools.partial(
        pl.kernel,
        mesh=mesh,
        out_type=jax.ShapeDtypeStruct((_NCORE, NP, D), jnp.float32),
        scratch_types=[
            pltpu.VMEM((_KC, D), jnp.float32),
            pltpu.VMEM((_KC, D), jnp.float32),
            pltpu.VMEM((_KC, D), jnp.float32),
            pltpu.VMEM((_KC, D), jnp.float32),
            pltpu.VMEM((8, _KC), jnp.int32),
            pltpu.VMEM((8, _KC), jnp.int32),
            pltpu.SemaphoreType.DMA,
            pltpu.SemaphoreType.DMA,
            pltpu.SemaphoreType.DMA,
            pltpu.SemaphoreType.DMA,
            pltpu.SemaphoreType.DMA,
            pltpu.SemaphoreType.DMA,
            pltpu.SemaphoreType.DMA,
            pltpu.SemaphoreType.DMA,
            pltpu.SemaphoreType.DMA,
            pltpu.SemaphoreType.DMA,
            pltpu.SemaphoreType.DMA,
            pltpu.SemaphoreType.DMA,
            pltpu.VMEM_SHARED((NP, D), jnp.float32),
        ],
    )
    def sc_propagate(table_hbm, src_hbm, dst_hbm, zeros_hbm, out_hbm,
                     r0, r1, r2, r3, sb, db,
                     i0, i1, i2, i3, i4, i5, i6, i7, g0, g1, g2, g3, acc):
        c = lax.axis_index("c")
        s = lax.axis_index("s")
        rows = (r0, r1, r2, r3)
        isem = (i0, i1, i2, i3, i4, i5, i6, i7)
        gsem = (g0, g1, g2, g3)

        base = (c * _NSUB + s) * _EPT

        def idx_load(j, u6):
            off = pl.multiple_of(base + j * _KC, 8)
            pltpu.async_copy(src_hbm.at[pl.ds(off, _KC)], sb.at[u6], isem[u6])
            pltpu.async_copy(dst_hbm.at[pl.ds(off, _KC)], db.at[u6], isem[u6])

        def idx_wait(u6):
            pltpu.make_async_copy(src_hbm.at[pl.ds(0, _KC)], sb.at[u6],
                                  isem[u6]).wait()
            pltpu.make_async_copy(dst_hbm.at[pl.ds(0, _KC)], db.at[u6],
                                  isem[u6]).wait()

        for m in range(8):
            idx_load(m, m)
        for m in range(3):
            idx_wait(m)
            pltpu.async_copy(table_hbm.at[sb.at[m]], rows[m], gsem[m])
        pltpu.sync_copy(zeros_hbm.at[pl.ds(s * _RPT, _RPT)],
                        acc.at[pl.ds(s * _RPT, _RPT)])
        plsc.subcore_barrier()

        def step(j, u4, u8):
            @pl.when(j + 3 < _NCH)
            def _launch():
                idx_wait((u8 + 3) % 8)
                pltpu.async_copy(table_hbm.at[sb.at[(u8 + 3) % 8]],
                                 rows[(u4 + 3) % 4], gsem[(u4 + 3) % 4])

            pltpu.make_async_copy(table_hbm.at[sb.at[u8]], rows[u4],
                                  gsem[u4]).wait()
            pltpu.sync_copy(rows[u4], acc.at[db.at[u8]], add=True)

            @pl.when(j + 8 < _NCH)
            def _next_idx():
                idx_load(j + 8, u8)

        def body(k, carry):
            j = pl.multiple_of(k * 8, 8)
            for u in range(8):
                step(j + u, u % 4, u)
            return carry

        nfull = _NCH // 8 - 1
        lax.fori_loop(0, nfull, body, 0)
        for j in range(nfull * 8, _NCH):
            step(j, j % 4, j % 8)
        plsc.subcore_barrier()
        pltpu.sync_copy(acc.at[pl.ds(s * _RPT, _RPT)],
                        out_hbm.at[c, pl.ds(s * _RPT, _RPT)])

    return sc_degree, sc_propagate


def _tc_layer1(x, W1, degp):
    def body(x_ref, w_ref, degp_ref, scaled_ref, dinv_ref):
        dp = degp_ref[...]
        deg = dp[0, :N] + dp[1, :N] + 1.0
        dinv = lax.rsqrt(deg)
        dinv_ref[...] = dinv
        hw = jnp.dot(x_ref[...], w_ref[...], preferred_element_type=jnp.float32)
        scaled_ref[...] = hw * dinv

    return pl.pallas_call(
        body,
        out_shape=(jax.ShapeDtypeStruct((N, D), jnp.float32),
                   jax.ShapeDtypeStruct((N, 1), jnp.float32)),
    )(x, W1, degp)


def _tc_mid(tp, scaled_prev, dinv, b_prev, W):
    def body(tp_ref, sc_ref, dinv_ref, b_ref, w_ref, out_ref):
        tp_ = tp_ref[...]
        t = tp_[0, :N] + tp_[1, :N]
        dinv_ = dinv_ref[...]
        h = jnp.maximum(dinv_ * (t + sc_ref[...]) + b_ref[...], 0.0)
        out_ref[...] = jnp.dot(h, w_ref[...],
                               preferred_element_type=jnp.float32) * dinv_

    return pl.pallas_call(
        body,
        out_shape=jax.ShapeDtypeStruct((N, D), jnp.float32),
    )(tp, scaled_prev, dinv, b_prev, W)


def _tc_final(tp, scaled_prev, dinv, b_prev, batch2d,
              Wm1, bm1, Wm2, bm2, Wh1, bh1, Wh2, bh2):
    def body(tp_ref, sc_ref, dinv_ref, b_ref, batch_ref,
             wm1_ref, bm1_ref, wm2_ref, bm2_ref,
             wh1_ref, bh1_ref, wh2_ref, bh2_ref, out_ref):
        tp_ = tp_ref[...]
        t = tp_[0, :N] + tp_[1, :N]
        h = dinv_ref[...] * (t + sc_ref[...]) + b_ref[...]
        gids = lax.broadcasted_iota(jnp.int32, (N, G), 1)
        onehot = (batch_ref[...] == gids).astype(jnp.float32)
        dn = (((0,), (0,)), ((), ()))
        sums = lax.dot_general(onehot, h, dn,
                               preferred_element_type=jnp.float32)
        counts = lax.dot_general(onehot, jnp.ones((N, 1), jnp.float32), dn,
                                 preferred_element_type=jnp.float32)
        pooled = sums / jnp.maximum(counts, 1.0)
        z = jnp.maximum(jnp.dot(pooled, wm1_ref[...],
                                preferred_element_type=jnp.float32)
                        + bm1_ref[...], 0.0)
        z = jnp.maximum(jnp.dot(z, wm2_ref[...],
                                preferred_element_type=jnp.float32)
                        + bm2_ref[...], 0.0)
        r = jnp.maximum(jnp.dot(z, wh1_ref[...],
                                preferred_element_type=jnp.float32)
                        + bh1_ref[...], 0.0)
        out_ref[...] = jnp.dot(r, wh2_ref[...],
                               preferred_element_type=jnp.float32) + bh2_ref[...]

    return pl.pallas_call(
        body,
        out_shape=jax.ShapeDtypeStruct((G, D), jnp.float32),
    )(tp, scaled_prev, dinv, b_prev, batch2d,
      Wm1, bm1, Wm2, bm2, Wh1, bh1, Wh2, bh2)


def kernel(x, edge_index, batch, W1, b1, W2, b2, W3, b3,
           Wm1, bm1, Wm2, bm2, Wh1, bh1, Wh2, bh2):
    sc_degree, sc_propagate = _sc_kernels()
    ntile = _NCORE * _NSUB
    ereal = E // ntile
    epad = _EPT - ereal
    pad = jnp.arange(epad, dtype=jnp.int32).reshape(1, epad)
    src2 = edge_index[0].astype(jnp.int32).reshape(ntile, ereal)
    dst2 = edge_index[1].astype(jnp.int32).reshape(ntile, ereal)
    src = jnp.concatenate([src2, jnp.broadcast_to(pad, (ntile, epad))], axis=1)
    dst = jnp.concatenate([dst2, jnp.broadcast_to(pad + N, (ntile, epad))],
                          axis=1)
    dst2d = dst.reshape(_EPAD // _KC, _KC)
    src = src.reshape(_EPAD)
    dst = dst.reshape(_EPAD)
    z1 = jnp.zeros((NP,), jnp.float32)
    z2 = jnp.zeros((NP, D), jnp.float32)

    degp = sc_degree(dst2d, z1).reshape(_NCORE, NP, 1)
    scaled1, dinv = _tc_layer1(x, W1, degp)
    t1 = sc_propagate(scaled1, src, dst, z2)
    scaled2 = _tc_mid(t1, scaled1, dinv, b1.reshape(1, D), W2)
    t2 = sc_propagate(scaled2, src, dst, z2)
    scaled3 = _tc_mid(t2, scaled2, dinv, b2.reshape(1, D), W3)
    t3 = sc_propagate(scaled3, src, dst, z2)
    return _tc_final(t3, scaled3, dinv, b3.reshape(1, D),
                     batch.astype(jnp.int32).reshape(N, 1),
                     Wm1, bm1.reshape(1, -1), Wm2, bm2.reshape(1, -1),
                     Wh1, bh1.reshape(1, -1), Wh2, bh2.reshape(1, -1))
